# Initial kernel scaffold; baseline (speedup 1.0000x reference)
import jax
import jax.numpy as jnp
from jax.experimental import pallas as pl


def kernel(x, edge_index, W1, b1, W2, b2):
    raise NotImplementedError("write your pallas kernel here")


# speedup vs baseline: 110.0604x; 110.0604x over previous
import functools

import jax
import jax.numpy as jnp
from jax import lax
from jax.experimental import pallas as pl
from jax.experimental.pallas import tpu as pltpu
from jax.experimental.pallas import tpu_sc as plsc

N = 100000
E = 6400000

NC = 2
NS = 16
NW = NC * NS

B = 128
KI = 8
CL = B * KI

NA = 102400
SL = NA // NS
ROWS = NA // 128

EW = 200704
EP = EW * NW
RW = EW // B
NBODY = EW // CL
EPR = EP // B

_MESH = dict(core_axis_name="c", subcore_axis_name="s", num_cores=NC,
             num_subcores=NS)


def _sc_deg(dst_rows, zinit):
    @functools.partial(
        pl.kernel,
        out_type=jax.ShapeDtypeStruct((NC, NA), jnp.float32),
        mesh=plsc.VectorSubcoreMesh(**_MESH),
        scratch_types=[
            pltpu.VMEM((KI, B), jnp.int32),
            pltpu.VMEM((B,), jnp.float32),
            pltpu.VMEM_SHARED((NA,), jnp.float32),
            pltpu.SemaphoreType.DMA,
        ],
    )
    def k(dst_h, z_h, out_h, idx_v, ones_v, acc, sem_s):
        c = lax.axis_index("c")
        s = lax.axis_index("s")
        w = c * NS + s
        for t in range(B // 16):
            ones_v[pl.ds(t * 16, 16)] = jnp.ones((16,), jnp.float32)
        pltpu.sync_copy(z_h.at[pl.ds(s * SL, SL)], acc.at[pl.ds(s * SL, SL)])
        plsc.subcore_barrier()

        def body(i, carry):
            rbase = w * RW + i * KI
            pltpu.sync_copy(dst_h.at[pl.ds(rbase, KI)], idx_v)
            descs = [
                pltpu.async_copy(ones_v, acc.at[idx_v.at[j]], sem_s, add=True)
                for j in range(KI)
            ]
            for d in descs:
                d.wait()
            return carry

        lax.fori_loop(0, NBODY, body, 0)
        plsc.subcore_barrier()
        pltpu.sync_copy(acc.at[pl.ds(s * SL, SL)],
                        out_h.at[c, pl.ds(s * SL, SL)])

    return k(dst_rows, zinit)


def _sc_agg(src_rows, dst_rows, tables, zinit):
    nt = len(tables)

    @functools.partial(
        pl.kernel,
        out_type=tuple(jax.ShapeDtypeStruct((NC, NA), jnp.float32)
                       for _ in range(nt)),
        mesh=plsc.VectorSubcoreMesh(**_MESH),
        scratch_types=(
            [pltpu.VMEM((KI, B), jnp.int32)] * 2
            + [pltpu.VMEM((KI, B), jnp.float32)] * nt
            + [pltpu.VMEM_SHARED((NA,), jnp.float32)] * nt
            + [pltpu.SemaphoreType.DMA] * 3
        ),
    )
    def k(src_h, dst_h, *rest):
        tabs_h = rest[:nt]
        z_h = rest[nt]
        outs_h = rest[nt + 1:2 * nt + 1]
        src_v, dst_v = rest[2 * nt + 1:2 * nt + 3]
        vals_v = rest[2 * nt + 3:3 * nt + 3]
        accs = rest[3 * nt + 3:4 * nt + 3]
        sem_l, sem_g, sem_s = rest[4 * nt + 3:]
        c = lax.axis_index("c")
        s = lax.axis_index("s")
        w = c * NS + s
        for a in accs:
            pltpu.sync_copy(z_h.at[pl.ds(s * SL, SL)], a.at[pl.ds(s * SL, SL)])
        plsc.subcore_barrier()

        def body(i, carry):
            rbase = w * RW + i * KI
            l1 = pltpu.async_copy(src_h.at[pl.ds(rbase, KI)], src_v, sem_l)
            l2 = pltpu.async_copy(dst_h.at[pl.ds(rbase, KI)], dst_v, sem_l)
            l1.wait()
            l2.wait()
            gd = [
                pltpu.async_copy(t.at[src_v.at[j]], v.at[j], sem_g)
                for t, v in zip(tabs_h, vals_v)
                for j in range(KI)
            ]
            for d in gd:
                d.wait()
            sd = [
                pltpu.async_copy(v.at[j], a.at[dst_v.at[j]], sem_s, add=True)
                for v, a in zip(vals_v, accs)
                for j in range(KI)
            ]
            for d in sd:
                d.wait()
            return carry

        lax.fori_loop(0, NBODY, body, 0)
        plsc.subcore_barrier()
        for a, o in zip(accs, outs_h):
            pltpu.sync_copy(a.at[pl.ds(s * SL, SL)],
                            o.at[c, pl.ds(s * SL, SL)])

    return k(src_rows, dst_rows, *tables, zinit)


def _tc_prep(degcm, xc):
    def body(deg_ref, xc_ref, dis_ref, u_ref):
        deg = deg_ref[0] + deg_ref[1] + 1.0
        dis = lax.rsqrt(deg)
        dis_ref[...] = dis
        u_ref[0] = dis * xc_ref[0]
        u_ref[1] = dis * xc_ref[1]

    return pl.pallas_call(
        body,
        out_shape=(jax.ShapeDtypeStruct((ROWS, 128), jnp.float32),
                   jax.ShapeDtypeStruct((2, ROWS, 128), jnp.float32)),
    )(degcm, xc)


def _tc_mid(s1a, s1b, ucm, discm, W1, b1, W2):
    def body(s1a_ref, s1b_ref, u_ref, dis_ref, w1_ref, b1_ref, w2_ref, v_ref):
        dis = dis_ref[...]
        za = dis * (s1a_ref[0] + s1a_ref[1] + u_ref[0])
        zb = dis * (s1b_ref[0] + s1b_ref[1] + u_ref[1])
        t = jnp.zeros((ROWS, 128), jnp.float32)
        for o in range(16):
            h = jnp.maximum(za * w1_ref[0, o] + zb * w1_ref[1, o] + b1_ref[o],
                            0.0)
            t = t + h * w2_ref[o, 0]
        v_ref[...] = dis * t

    vspec = pl.BlockSpec(memory_space=pltpu.VMEM)
    sspec = pl.BlockSpec(memory_space=pltpu.SMEM)
    return pl.pallas_call(
        body,
        in_specs=[vspec, vspec, vspec, vspec, sspec, sspec, sspec],
        out_specs=vspec,
        out_shape=jax.ShapeDtypeStruct((ROWS, 128), jnp.float32),
    )(s1a, s1b, ucm, discm, W1, b1, W2)


def _tc_fin(s2cm, vcm, discm, b2):
    def body(s2_ref, v_ref, dis_ref, b2_ref, o_ref):
        o_ref[...] = (dis_ref[...] * (s2_ref[0] + s2_ref[1] + v_ref[...])
                      + b2_ref[0])

    vspec = pl.BlockSpec(memory_space=pltpu.VMEM)
    sspec = pl.BlockSpec(memory_space=pltpu.SMEM)
    return pl.pallas_call(
        body,
        in_specs=[vspec, vspec, vspec, sspec],
        out_specs=vspec,
        out_shape=jax.ShapeDtypeStruct((ROWS, 128), jnp.float32),
    )(s2cm, vcm, discm, b2)


def kernel(x, edge_index, W1, b1, W2, b2):
    ei = edge_index.astype(jnp.int32)
    npad = EP - E
    padv = (jnp.arange(npad, dtype=jnp.int32) % 2048) + N
    src_rows = jnp.concatenate([ei[0], padv]).reshape(EPR, B)
    dst_rows = jnp.concatenate([ei[1], padv]).reshape(EPR, B)

    xp = jnp.pad(x, ((0, NA - N), (0, 0)))
    xc = xp.T.reshape(2, ROWS, 128)
    z1 = jnp.zeros((NA,), jnp.float32)

    degp = _sc_deg(dst_rows, z1)
    discm, ucm = _tc_prep(degp.reshape(NC, ROWS, 128), xc)

    u0_tab = ucm[0].reshape(NA)
    u1_tab = ucm[1].reshape(NA)
    s1p0, s1p1 = _sc_agg(src_rows, dst_rows, (u0_tab, u1_tab), z1)
    vcm = _tc_mid(s1p0.reshape(NC, ROWS, 128), s1p1.reshape(NC, ROWS, 128),
                  ucm, discm, W1, b1, W2)

    v_tab = vcm.reshape(NA)
    (s2p,) = _sc_agg(src_rows, dst_rows, (v_tab,), z1)
    outcm = _tc_fin(s2p.reshape(NC, ROWS, 128), vcm, discm, b2)
    return outcm.reshape(NA)[:N]


# speedup vs baseline: 193.9952x; 1.7626x over previous
import functools

import jax
import jax.numpy as jnp
from jax import lax
from jax.experimental import pallas as pl
from jax.experimental.pallas import tpu as pltpu
from jax.experimental.pallas import tpu_sc as plsc

N = 100000
E = 6400000

NC = 2
NS = 16
NW = NC * NS

B = 128
KI = 8
CL = B * KI

NA = 102400
SL = NA // NS
ROWS = NA // 128

EW = 200704
EP = EW * NW
RW = EW // B
NBODY = EW // CL
EPR = EP // B

_MESH = dict(core_axis_name="c", subcore_axis_name="s", num_cores=NC,
             num_subcores=NS)


def _sc_deg(dst_rows, zinit):
    @functools.partial(
        pl.kernel,
        out_type=jax.ShapeDtypeStruct((NC, NA), jnp.float32),
        mesh=plsc.VectorSubcoreMesh(**_MESH),
        scratch_types=[
            pltpu.VMEM((KI, B), jnp.int32),
            pltpu.VMEM((B,), jnp.float32),
            pltpu.VMEM_SHARED((NA,), jnp.float32),
            pltpu.SemaphoreType.DMA,
        ],
    )
    def k(dst_h, z_h, out_h, idx_v, ones_v, acc, sem_s):
        c = lax.axis_index("c")
        s = lax.axis_index("s")
        w = c * NS + s
        for t in range(B // 16):
            ones_v[pl.ds(t * 16, 16)] = jnp.ones((16,), jnp.float32)
        pltpu.sync_copy(z_h.at[pl.ds(s * SL, SL)], acc.at[pl.ds(s * SL, SL)])
        plsc.subcore_barrier()

        def body(i, carry):
            rbase = w * RW + i * KI
            pltpu.sync_copy(dst_h.at[pl.ds(rbase, KI)], idx_v)
            descs = [
                pltpu.async_copy(ones_v, acc.at[idx_v.at[j]], sem_s, add=True)
                for j in range(KI)
            ]
            for d in descs:
                d.wait()
            return carry

        lax.fori_loop(0, NBODY, body, 0)
        plsc.subcore_barrier()
        pltpu.sync_copy(acc.at[pl.ds(s * SL, SL)],
                        out_h.at[c, pl.ds(s * SL, SL)])

    return k(dst_rows, zinit)


def _sc_agg1(src_rows, dst_rows, table, zinit):
    @functools.partial(
        pl.kernel,
        out_type=jax.ShapeDtypeStruct((NC, NA), jnp.float32),
        mesh=plsc.VectorSubcoreMesh(**_MESH),
        scratch_types=[
            pltpu.VMEM((NA,), jnp.float32),
            pltpu.VMEM((KI, B), jnp.int32),
            pltpu.VMEM((KI, B), jnp.int32),
            pltpu.VMEM((KI, B), jnp.float32),
            pltpu.VMEM_SHARED((NA,), jnp.float32),
            pltpu.SemaphoreType.DMA,
            pltpu.SemaphoreType.DMA,
        ],
        compiler_params=pltpu.CompilerParams(needs_layout_passes=False),
    )
    def k(src_h, dst_h, tab_h, z_h, out_h, tab_v, src_v, dst_v, vals_v, acc,
          sem_l, sem_s):
        c = lax.axis_index("c")
        s = lax.axis_index("s")
        w = c * NS + s
        pltpu.sync_copy(tab_h, tab_v)
        pltpu.sync_copy(z_h.at[pl.ds(s * SL, SL)], acc.at[pl.ds(s * SL, SL)])
        plsc.subcore_barrier()

        def body(i, carry):
            rbase = w * RW + i * KI
            l1 = pltpu.async_copy(src_h.at[pl.ds(rbase, KI)], src_v, sem_l)
            l2 = pltpu.async_copy(dst_h.at[pl.ds(rbase, KI)], dst_v, sem_l)
            l1.wait()
            l2.wait()
            sd = []
            for j in range(KI):
                for t in range(B // 16):
                    sl = (j, pl.ds(t * 16, 16))
                    vals_v[sl] = plsc.load_gather(tab_v, [src_v[sl]])
                sd.append(pltpu.async_copy(vals_v.at[j],
                                           acc.at[dst_v.at[j]], sem_s,
                                           add=True))
            for d in sd:
                d.wait()
            return carry

        lax.fori_loop(0, NBODY, body, 0)
        plsc.subcore_barrier()
        pltpu.sync_copy(acc.at[pl.ds(s * SL, SL)],
                        out_h.at[c, pl.ds(s * SL, SL)])

    return k(src_rows, dst_rows, table, zinit)


EW2 = EP // 16
RW2 = EW2 // B
NBODY2 = RW2 // KI
SL2 = 2 * NA // NS


def _sc_agg2(src_rows, dst_rows, table2, zinit):
    @functools.partial(
        pl.kernel,
        out_type=jax.ShapeDtypeStruct((NC, 2 * NA), jnp.float32),
        mesh=plsc.VectorSubcoreMesh(**_MESH),
        scratch_types=[
            pltpu.VMEM((NA,), jnp.float32),
            pltpu.VMEM((KI, B), jnp.int32),
            pltpu.VMEM((KI, B), jnp.int32),
            pltpu.VMEM((KI, B), jnp.float32),
            pltpu.VMEM_SHARED((2 * NA,), jnp.float32),
            pltpu.SemaphoreType.DMA,
            pltpu.SemaphoreType.DMA,
        ],
        compiler_params=pltpu.CompilerParams(needs_layout_passes=False),
    )
    def k(src_h, dst_h, tab_h, z_h, out_h, tab_v, src_v, dst_v, vals_v, acc,
          sem_l, sem_s):
        c = lax.axis_index("c")
        s = lax.axis_index("s")
        ch = s % 2
        chunk = c * 8 + s // 2
        offs = ch * NA
        pltpu.sync_copy(tab_h.at[pl.ds(ch * NA, NA)], tab_v)
        pltpu.sync_copy(z_h.at[pl.ds(0, SL2)], acc.at[pl.ds(s * SL2, SL2)])
        plsc.subcore_barrier()

        def body(i, carry):
            rbase = chunk * RW2 + i * KI
            l1 = pltpu.async_copy(src_h.at[pl.ds(rbase, KI)], src_v, sem_l)
            l2 = pltpu.async_copy(dst_h.at[pl.ds(rbase, KI)], dst_v, sem_l)
            l1.wait()
            l2.wait()
            sd = []
            for j in range(KI):
                for t in range(B // 16):
                    sl = (j, pl.ds(t * 16, 16))
                    dst_v[sl] = dst_v[sl] + jnp.full((16,), offs, jnp.int32)
                    vals_v[sl] = plsc.load_gather(tab_v, [src_v[sl]])
                sd.append(pltpu.async_copy(vals_v.at[j],
                                           acc.at[dst_v.at[j]], sem_s,
                                           add=True))
            for d in sd:
                d.wait()
            return carry

        lax.fori_loop(0, NBODY2, body, 0)
        plsc.subcore_barrier()
        pltpu.sync_copy(acc.at[pl.ds(s * SL2, SL2)],
                        out_h.at[c, pl.ds(s * SL2, SL2)])

    return k(src_rows, dst_rows, table2, zinit)


def _tc_prep(degcm, xc):
    def body(deg_ref, xc_ref, dis_ref, u_ref):
        deg = deg_ref[0] + deg_ref[1] + 1.0
        dis = lax.rsqrt(deg)
        dis_ref[...] = dis
        u_ref[0] = dis * xc_ref[0]
        u_ref[1] = dis * xc_ref[1]

    return pl.pallas_call(
        body,
        out_shape=(jax.ShapeDtypeStruct((ROWS, 128), jnp.float32),
                   jax.ShapeDtypeStruct((2, ROWS, 128), jnp.float32)),
    )(degcm, xc)


def _tc_mid(s1a, s1b, ucm, discm, W1, b1, W2):
    def body(s1a_ref, s1b_ref, u_ref, dis_ref, w1_ref, b1_ref, w2_ref, v_ref):
        dis = dis_ref[...]
        za = dis * (s1a_ref[0] + s1a_ref[1] + u_ref[0])
        zb = dis * (s1b_ref[0] + s1b_ref[1] + u_ref[1])
        t = jnp.zeros((ROWS, 128), jnp.float32)
        for o in range(16):
            h = jnp.maximum(za * w1_ref[0, o] + zb * w1_ref[1, o] + b1_ref[o],
                            0.0)
            t = t + h * w2_ref[o, 0]
        v_ref[...] = dis * t

    vspec = pl.BlockSpec(memory_space=pltpu.VMEM)
    sspec = pl.BlockSpec(memory_space=pltpu.SMEM)
    return pl.pallas_call(
        body,
        in_specs=[vspec, vspec, vspec, vspec, sspec, sspec, sspec],
        out_specs=vspec,
        out_shape=jax.ShapeDtypeStruct((ROWS, 128), jnp.float32),
    )(s1a, s1b, ucm, discm, W1, b1, W2)


def _tc_fin(s2cm, vcm, discm, b2):
    def body(s2_ref, v_ref, dis_ref, b2_ref, o_ref):
        o_ref[...] = (dis_ref[...] * (s2_ref[0] + s2_ref[1] + v_ref[...])
                      + b2_ref[0])

    vspec = pl.BlockSpec(memory_space=pltpu.VMEM)
    sspec = pl.BlockSpec(memory_space=pltpu.SMEM)
    return pl.pallas_call(
        body,
        in_specs=[vspec, vspec, vspec, sspec],
        out_specs=vspec,
        out_shape=jax.ShapeDtypeStruct((ROWS, 128), jnp.float32),
    )(s2cm, vcm, discm, b2)


def kernel(x, edge_index, W1, b1, W2, b2):
    ei = edge_index.astype(jnp.int32)
    npad = EP - E
    padv = (jnp.arange(npad, dtype=jnp.int32) % 2048) + N
    src_rows = jnp.concatenate([ei[0], padv]).reshape(EPR, B)
    dst_rows = jnp.concatenate([ei[1], padv]).reshape(EPR, B)

    xp = jnp.pad(x, ((0, NA - N), (0, 0)))
    xc = xp.T.reshape(2, ROWS, 128)
    z1 = jnp.zeros((NA,), jnp.float32)

    degp = _sc_deg(dst_rows, z1)
    discm, ucm = _tc_prep(degp.reshape(NC, ROWS, 128), xc)

    u_flat = ucm.reshape(2 * NA)
    s1p = _sc_agg2(src_rows, dst_rows, u_flat, z1)
    s1f = s1p.reshape(NC, 2, ROWS, 128)
    vcm = _tc_mid(s1f[:, 0], s1f[:, 1],
                  ucm, discm, W1, b1, W2)

    v_tab = vcm.reshape(NA)
    s2p = _sc_agg1(src_rows, dst_rows, v_tab, z1)
    outcm = _tc_fin(s2p.reshape(NC, ROWS, 128), vcm, discm, b2)
    return outcm.reshape(NA)[:N]


# speedup vs baseline: 255.2470x; 1.3157x over previous
import functools

import jax
import jax.numpy as jnp
from jax import lax
from jax.experimental import pallas as pl
from jax.experimental.pallas import tpu as pltpu
from jax.experimental.pallas import tpu_sc as plsc

N = 100000
E = 6400000

NC = 2
NS = 16
NW = NC * NS

B = 128
KI = 16
CL = B * KI

NA = 102400
SL = NA // NS
ROWS = NA // 128

EW = 200704
EP = EW * NW
RW = EW // B
NBODY = EW // CL
EPR = EP // B

_MESH = dict(core_axis_name="c", subcore_axis_name="s", num_cores=NC,
             num_subcores=NS)


def _sc_deg(dst_rows, zinit):
    @functools.partial(
        pl.kernel,
        out_type=jax.ShapeDtypeStruct((NC, NA), jnp.float32),
        mesh=plsc.VectorSubcoreMesh(**_MESH),
        scratch_types=[
            pltpu.VMEM((KI, B), jnp.int32),
            pltpu.VMEM((B,), jnp.float32),
            pltpu.VMEM_SHARED((NA,), jnp.float32),
            pltpu.SemaphoreType.DMA,
        ],
    )
    def k(dst_h, z_h, out_h, idx_v, ones_v, acc, sem_s):
        c = lax.axis_index("c")
        s = lax.axis_index("s")
        w = c * NS + s
        for t in range(B // 16):
            ones_v[pl.ds(t * 16, 16)] = jnp.ones((16,), jnp.float32)
        pltpu.sync_copy(z_h.at[pl.ds(s * SL, SL)], acc.at[pl.ds(s * SL, SL)])
        plsc.subcore_barrier()

        def body(i, carry):
            rbase = w * RW + i * KI
            pltpu.sync_copy(dst_h.at[pl.ds(rbase, KI)], idx_v)
            descs = [
                pltpu.async_copy(ones_v, acc.at[idx_v.at[j]], sem_s, add=True)
                for j in range(KI)
            ]
            for d in descs:
                d.wait()
            return carry

        lax.fori_loop(0, NBODY, body, 0)
        plsc.subcore_barrier()
        pltpu.sync_copy(acc.at[pl.ds(s * SL, SL)],
                        out_h.at[c, pl.ds(s * SL, SL)])

    return k(dst_rows, zinit)


def _sc_agg1(src_rows, dst_rows, table, zinit):
    @functools.partial(
        pl.kernel,
        out_type=jax.ShapeDtypeStruct((NC, NA), jnp.float32),
        mesh=plsc.VectorSubcoreMesh(**_MESH),
        scratch_types=[
            pltpu.VMEM((NA,), jnp.float32),
            pltpu.VMEM((KI, B), jnp.int32),
            pltpu.VMEM((KI, B), jnp.int32),
            pltpu.VMEM((KI, B), jnp.float32),
            pltpu.VMEM_SHARED((NA,), jnp.float32),
            pltpu.SemaphoreType.DMA,
            pltpu.SemaphoreType.DMA,
        ],
        compiler_params=pltpu.CompilerParams(needs_layout_passes=False),
    )
    def k(src_h, dst_h, tab_h, z_h, out_h, tab_v, src_v, dst_v, vals_v, acc,
          sem_l, sem_s):
        c = lax.axis_index("c")
        s = lax.axis_index("s")
        w = c * NS + s
        pltpu.sync_copy(tab_h, tab_v)
        pltpu.sync_copy(z_h.at[pl.ds(s * SL, SL)], acc.at[pl.ds(s * SL, SL)])
        plsc.subcore_barrier()

        def body(i, carry):
            rbase = w * RW + i * KI
            l1 = pltpu.async_copy(src_h.at[pl.ds(rbase, KI)], src_v, sem_l)
            l2 = pltpu.async_copy(dst_h.at[pl.ds(rbase, KI)], dst_v, sem_l)
            l1.wait()
            l2.wait()
            sd = []
            for j in range(KI):
                for t in range(B // 16):
                    sl = (j, pl.ds(t * 16, 16))
                    vals_v[sl] = plsc.load_gather(tab_v, [src_v[sl]])
                sd.append(pltpu.async_copy(vals_v.at[j],
                                           acc.at[dst_v.at[j]], sem_s,
                                           add=True))
            for d in sd:
                d.wait()
            return carry

        lax.fori_loop(0, NBODY, body, 0)
        plsc.subcore_barrier()
        pltpu.sync_copy(acc.at[pl.ds(s * SL, SL)],
                        out_h.at[c, pl.ds(s * SL, SL)])

    return k(src_rows, dst_rows, table, zinit)


EW2 = EP // 16
RW2 = EW2 // B
NBODY2 = RW2 // KI
SL2 = 2 * NA // NS


def _sc_agg2(src_rows, dst_rows, table2, zinit):
    @functools.partial(
        pl.kernel,
        out_type=jax.ShapeDtypeStruct((NC, 2 * NA), jnp.float32),
        mesh=plsc.VectorSubcoreMesh(**_MESH),
        scratch_types=[
            pltpu.VMEM((NA,), jnp.float32),
            pltpu.VMEM((KI, B), jnp.int32),
            pltpu.VMEM((KI, B), jnp.int32),
            pltpu.VMEM((KI, B), jnp.float32),
            pltpu.VMEM_SHARED((2 * NA,), jnp.float32),
            pltpu.SemaphoreType.DMA,
            pltpu.SemaphoreType.DMA,
        ],
        compiler_params=pltpu.CompilerParams(needs_layout_passes=False),
    )
    def k(src_h, dst_h, tab_h, z_h, out_h, tab_v, src_v, dst_v, vals_v, acc,
          sem_l, sem_s):
        c = lax.axis_index("c")
        s = lax.axis_index("s")
        ch = s % 2
        chunk = c * 8 + s // 2
        offs = ch * NA
        pltpu.sync_copy(tab_h.at[pl.ds(ch * NA, NA)], tab_v)
        pltpu.sync_copy(z_h.at[pl.ds(0, SL2)], acc.at[pl.ds(s * SL2, SL2)])
        plsc.subcore_barrier()

        def body(i, carry):
            rbase = chunk * RW2 + i * KI
            l1 = pltpu.async_copy(src_h.at[pl.ds(rbase, KI)], src_v, sem_l)
            l2 = pltpu.async_copy(dst_h.at[pl.ds(rbase, KI)], dst_v, sem_l)
            l1.wait()
            l2.wait()
            sd = []
            for j in range(KI):
                for t in range(B // 16):
                    sl = (j, pl.ds(t * 16, 16))
                    dst_v[sl] = dst_v[sl] + jnp.full((16,), offs, jnp.int32)
                    vals_v[sl] = plsc.load_gather(tab_v, [src_v[sl]])
                sd.append(pltpu.async_copy(vals_v.at[j],
                                           acc.at[dst_v.at[j]], sem_s,
                                           add=True))
            for d in sd:
                d.wait()
            return carry

        lax.fori_loop(0, NBODY2, body, 0)
        plsc.subcore_barrier()
        pltpu.sync_copy(acc.at[pl.ds(s * SL2, SL2)],
                        out_h.at[c, pl.ds(s * SL2, SL2)])

    return k(src_rows, dst_rows, table2, zinit)


def _tc_prep(degcm, xc):
    def body(deg_ref, xc_ref, dis_ref, u_ref):
        deg = deg_ref[0] + deg_ref[1] + 1.0
        dis = lax.rsqrt(deg)
        dis_ref[...] = dis
        u_ref[0] = dis * xc_ref[0]
        u_ref[1] = dis * xc_ref[1]

    return pl.pallas_call(
        body,
        out_shape=(jax.ShapeDtypeStruct((ROWS, 128), jnp.float32),
                   jax.ShapeDtypeStruct((2, ROWS, 128), jnp.float32)),
    )(degcm, xc)


def _tc_mid(s1a, s1b, ucm, discm, W1, b1, W2):
    def body(s1a_ref, s1b_ref, u_ref, dis_ref, w1_ref, b1_ref, w2_ref, v_ref):
        dis = dis_ref[...]
        za = dis * (s1a_ref[0] + s1a_ref[1] + u_ref[0])
        zb = dis * (s1b_ref[0] + s1b_ref[1] + u_ref[1])
        t = jnp.zeros((ROWS, 128), jnp.float32)
        for o in range(16):
            h = jnp.maximum(za * w1_ref[0, o] + zb * w1_ref[1, o] + b1_ref[o],
                            0.0)
            t = t + h * w2_ref[o, 0]
        v_ref[...] = dis * t

    vspec = pl.BlockSpec(memory_space=pltpu.VMEM)
    sspec = pl.BlockSpec(memory_space=pltpu.SMEM)
    return pl.pallas_call(
        body,
        in_specs=[vspec, vspec, vspec, vspec, sspec, sspec, sspec],
        out_specs=vspec,
        out_shape=jax.ShapeDtypeStruct((ROWS, 128), jnp.float32),
    )(s1a, s1b, ucm, discm, W1, b1, W2)


def _tc_fin(s2cm, vcm, discm, b2):
    def body(s2_ref, v_ref, dis_ref, b2_ref, o_ref):
        o_ref[...] = (dis_ref[...] * (s2_ref[0] + s2_ref[1] + v_ref[...])
                      + b2_ref[0])

    vspec = pl.BlockSpec(memory_space=pltpu.VMEM)
    sspec = pl.BlockSpec(memory_space=pltpu.SMEM)
    return pl.pallas_call(
        body,
        in_specs=[vspec, vspec, vspec, sspec],
        out_specs=vspec,
        out_shape=jax.ShapeDtypeStruct((ROWS, 128), jnp.float32),
    )(s2cm, vcm, discm, b2)


def kernel(x, edge_index, W1, b1, W2, b2):
    ei = edge_index.astype(jnp.int32)
    npad = EP - E
    padv = (jnp.arange(npad, dtype=jnp.int32) % 2048) + N
    src_rows = jnp.concatenate([ei[0], padv]).reshape(EPR, B)
    dst_rows = jnp.concatenate([ei[1], padv]).reshape(EPR, B)

    xp = jnp.pad(x, ((0, NA - N), (0, 0)))
    xc = xp.T.reshape(2, ROWS, 128)
    z1 = jnp.zeros((NA,), jnp.float32)

    degp = _sc_deg(dst_rows, z1)
    discm, ucm = _tc_prep(degp.reshape(NC, ROWS, 128), xc)

    u_flat = ucm.reshape(2 * NA)
    s1p = _sc_agg2(src_rows, dst_rows, u_flat, z1)
    s1f = s1p.reshape(NC, 2, ROWS, 128)
    vcm = _tc_mid(s1f[:, 0], s1f[:, 1],
                  ucm, discm, W1, b1, W2)

    v_tab = vcm.reshape(NA)
    s2p = _sc_agg1(src_rows, dst_rows, v_tab, z1)
    outcm = _tc_fin(s2p.reshape(NC, ROWS, 128), vcm, discm, b2)
    return outcm.reshape(NA)[:N]


# speedup vs baseline: 302.6815x; 1.1858x over previous
import functools

import jax
import jax.numpy as jnp
from jax import lax
from jax.experimental import pallas as pl
from jax.experimental.pallas import tpu as pltpu
from jax.experimental.pallas import tpu_sc as plsc

N = 100000
E = 6400000

NC = 2
NS = 16
NW = NC * NS

B = 128
KI = 32
CL = B * KI

NA = 102400
SL = NA // NS
ROWS = NA // 128

EW = 200704
EP = EW * NW
RW = EW // B
NBODY = EW // CL
EPR = EP // B

_MESH = dict(core_axis_name="c", subcore_axis_name="s", num_cores=NC,
             num_subcores=NS)


def _sc_deg(dst_rows, zinit):
    @functools.partial(
        pl.kernel,
        out_type=jax.ShapeDtypeStruct((NC, NA), jnp.float32),
        mesh=plsc.VectorSubcoreMesh(**_MESH),
        scratch_types=[
            pltpu.VMEM((KI, B), jnp.int32),
            pltpu.VMEM((B,), jnp.float32),
            pltpu.VMEM_SHARED((NA,), jnp.float32),
            pltpu.SemaphoreType.DMA,
        ],
    )
    def k(dst_h, z_h, out_h, idx_v, ones_v, acc, sem_s):
        c = lax.axis_index("c")
        s = lax.axis_index("s")
        w = c * NS + s
        for t in range(B // 16):
            ones_v[pl.ds(t * 16, 16)] = jnp.ones((16,), jnp.float32)
        pltpu.sync_copy(z_h.at[pl.ds(s * SL, SL)], acc.at[pl.ds(s * SL, SL)])
        plsc.subcore_barrier()

        def body(i, carry):
            rbase = w * RW + i * KI
            pltpu.sync_copy(dst_h.at[pl.ds(rbase, KI)], idx_v)
            descs = [
                pltpu.async_copy(ones_v, acc.at[idx_v.at[j]], sem_s, add=True)
                for j in range(KI)
            ]
            for d in descs:
                d.wait()
            return carry

        lax.fori_loop(0, NBODY, body, 0)
        plsc.subcore_barrier()
        pltpu.sync_copy(acc.at[pl.ds(s * SL, SL)],
                        out_h.at[c, pl.ds(s * SL, SL)])

    return k(dst_rows, zinit)


def _sc_agg1(src_rows, dst_rows, table, zinit):
    @functools.partial(
        pl.kernel,
        out_type=jax.ShapeDtypeStruct((NC, NA), jnp.float32),
        mesh=plsc.VectorSubcoreMesh(**_MESH),
        scratch_types=[
            pltpu.VMEM((NA,), jnp.float32),
            pltpu.VMEM((KI, B), jnp.int32),
            pltpu.VMEM((KI, B), jnp.int32),
            pltpu.VMEM((KI, B), jnp.float32),
            pltpu.VMEM_SHARED((NA,), jnp.float32),
            pltpu.SemaphoreType.DMA,
            pltpu.SemaphoreType.DMA,
        ],
        compiler_params=pltpu.CompilerParams(needs_layout_passes=False),
    )
    def k(src_h, dst_h, tab_h, z_h, out_h, tab_v, src_v, dst_v, vals_v, acc,
          sem_l, sem_s):
        c = lax.axis_index("c")
        s = lax.axis_index("s")
        w = c * NS + s
        pltpu.sync_copy(tab_h, tab_v)
        pltpu.sync_copy(z_h.at[pl.ds(s * SL, SL)], acc.at[pl.ds(s * SL, SL)])
        plsc.subcore_barrier()

        def body(i, carry):
            rbase = w * RW + i * KI
            l1 = pltpu.async_copy(src_h.at[pl.ds(rbase, KI)], src_v, sem_l)
            l2 = pltpu.async_copy(dst_h.at[pl.ds(rbase, KI)], dst_v, sem_l)
            l1.wait()
            l2.wait()
            sd = []
            for j in range(KI):
                for t in range(B // 16):
                    sl = (j, pl.ds(t * 16, 16))
                    vals_v[sl] = plsc.load_gather(tab_v, [src_v[sl]])
                sd.append(pltpu.async_copy(vals_v.at[j],
                                           acc.at[dst_v.at[j]], sem_s,
                                           add=True))
            for d in sd:
                d.wait()
            return carry

        lax.fori_loop(0, NBODY, body, 0)
        plsc.subcore_barrier()
        pltpu.sync_copy(acc.at[pl.ds(s * SL, SL)],
                        out_h.at[c, pl.ds(s * SL, SL)])

    return k(src_rows, dst_rows, table, zinit)


EW2 = EP // 16
RW2 = EW2 // B
NBODY2 = RW2 // KI
SL2 = 2 * NA // NS


def _sc_agg2(src_rows, dst_rows, table2, zinit):
    @functools.partial(
        pl.kernel,
        out_type=jax.ShapeDtypeStruct((NC, 2 * NA), jnp.float32),
        mesh=plsc.VectorSubcoreMesh(**_MESH),
        scratch_types=[
            pltpu.VMEM((NA,), jnp.float32),
            pltpu.VMEM((KI, B), jnp.int32),
            pltpu.VMEM((KI, B), jnp.int32),
            pltpu.VMEM((KI, B), jnp.float32),
            pltpu.VMEM_SHARED((2 * NA,), jnp.float32),
            pltpu.SemaphoreType.DMA,
            pltpu.SemaphoreType.DMA,
        ],
        compiler_params=pltpu.CompilerParams(needs_layout_passes=False),
    )
    def k(src_h, dst_h, tab_h, z_h, out_h, tab_v, src_v, dst_v, vals_v, acc,
          sem_l, sem_s):
        c = lax.axis_index("c")
        s = lax.axis_index("s")
        ch = s % 2
        chunk = c * 8 + s // 2
        offs = ch * NA
        pltpu.sync_copy(tab_h.at[pl.ds(ch * NA, NA)], tab_v)
        pltpu.sync_copy(z_h.at[pl.ds(0, SL2)], acc.at[pl.ds(s * SL2, SL2)])
        plsc.subcore_barrier()

        def body(i, carry):
            rbase = chunk * RW2 + i * KI
            l1 = pltpu.async_copy(src_h.at[pl.ds(rbase, KI)], src_v, sem_l)
            l2 = pltpu.async_copy(dst_h.at[pl.ds(rbase, KI)], dst_v, sem_l)
            l1.wait()
            l2.wait()
            sd = []
            for j in range(KI):
                for t in range(B // 16):
                    sl = (j, pl.ds(t * 16, 16))
                    dst_v[sl] = dst_v[sl] + jnp.full((16,), offs, jnp.int32)
                    vals_v[sl] = plsc.load_gather(tab_v, [src_v[sl]])
                sd.append(pltpu.async_copy(vals_v.at[j],
                                           acc.at[dst_v.at[j]], sem_s,
                                           add=True))
            for d in sd:
                d.wait()
            return carry

        lax.fori_loop(0, NBODY2, body, 0)
        plsc.subcore_barrier()
        pltpu.sync_copy(acc.at[pl.ds(s * SL2, SL2)],
                        out_h.at[c, pl.ds(s * SL2, SL2)])

    return k(src_rows, dst_rows, table2, zinit)


def _tc_prep(degcm, xc):
    def body(deg_ref, xc_ref, dis_ref, u_ref):
        deg = deg_ref[0] + deg_ref[1] + 1.0
        dis = lax.rsqrt(deg)
        dis_ref[...] = dis
        u_ref[0] = dis * xc_ref[0]
        u_ref[1] = dis * xc_ref[1]

    return pl.pallas_call(
        body,
        out_shape=(jax.ShapeDtypeStruct((ROWS, 128), jnp.float32),
                   jax.ShapeDtypeStruct((2, ROWS, 128), jnp.float32)),
    )(degcm, xc)


def _tc_mid(s1a, s1b, ucm, discm, W1, b1, W2):
    def body(s1a_ref, s1b_ref, u_ref, dis_ref, w1_ref, b1_ref, w2_ref, v_ref):
        dis = dis_ref[...]
        za = dis * (s1a_ref[0] + s1a_ref[1] + u_ref[0])
        zb = dis * (s1b_ref[0] + s1b_ref[1] + u_ref[1])
        t = jnp.zeros((ROWS, 128), jnp.float32)
        for o in range(16):
            h = jnp.maximum(za * w1_ref[0, o] + zb * w1_ref[1, o] + b1_ref[o],
                            0.0)
            t = t + h * w2_ref[o, 0]
        v_ref[...] = dis * t

    vspec = pl.BlockSpec(memory_space=pltpu.VMEM)
    sspec = pl.BlockSpec(memory_space=pltpu.SMEM)
    return pl.pallas_call(
        body,
        in_specs=[vspec, vspec, vspec, vspec, sspec, sspec, sspec],
        out_specs=vspec,
        out_shape=jax.ShapeDtypeStruct((ROWS, 128), jnp.float32),
    )(s1a, s1b, ucm, discm, W1, b1, W2)


def _tc_fin(s2cm, vcm, discm, b2):
    def body(s2_ref, v_ref, dis_ref, b2_ref, o_ref):
        o_ref[...] = (dis_ref[...] * (s2_ref[0] + s2_ref[1] + v_ref[...])
                      + b2_ref[0])

    vspec = pl.BlockSpec(memory_space=pltpu.VMEM)
    sspec = pl.BlockSpec(memory_space=pltpu.SMEM)
    return pl.pallas_call(
        body,
        in_specs=[vspec, vspec, vspec, sspec],
        out_specs=vspec,
        out_shape=jax.ShapeDtypeStruct((ROWS, 128), jnp.float32),
    )(s2cm, vcm, discm, b2)


def kernel(x, edge_index, W1, b1, W2, b2):
    ei = edge_index.astype(jnp.int32)
    npad = EP - E
    padv = (jnp.arange(npad, dtype=jnp.int32) % 2048) + N
    src_rows = jnp.concatenate([ei[0], padv]).reshape(EPR, B)
    dst_rows = jnp.concatenate([ei[1], padv]).reshape(EPR, B)

    xp = jnp.pad(x, ((0, NA - N), (0, 0)))
    xc = xp.T.reshape(2, ROWS, 128)
    z1 = jnp.zeros((NA,), jnp.float32)

    degp = _sc_deg(dst_rows, z1)
    discm, ucm = _tc_prep(degp.reshape(NC, ROWS, 128), xc)

    u_flat = ucm.reshape(2 * NA)
    s1p = _sc_agg2(src_rows, dst_rows, u_flat, z1)
    s1f = s1p.reshape(NC, 2, ROWS, 128)
    vcm = _tc_mid(s1f[:, 0], s1f[:, 1],
                  ucm, discm, W1, b1, W2)

    v_tab = vcm.reshape(NA)
    s2p = _sc_agg1(src_rows, dst_rows, v_tab, z1)
    outcm = _tc_fin(s2p.reshape(NC, ROWS, 128), vcm, discm, b2)
    return outcm.reshape(NA)[:N]


# speedup vs baseline: 367.2681x; 1.2134x over previous
import functools

import jax
import jax.numpy as jnp
from jax import lax
from jax.experimental import pallas as pl
from jax.experimental.pallas import tpu as pltpu
from jax.experimental.pallas import tpu_sc as plsc

N = 100000
E = 6400000

NC = 2
NS = 16
NW = NC * NS

B = 128
KI = 32
CL = B * KI

NA = 102400
SL = NA // NS
ROWS = NA // 128

EW = 200704
EP = EW * NW
RW = EW // B
NBODY = EW // CL
EPR = EP // B

_MESH = dict(core_axis_name="c", subcore_axis_name="s", num_cores=NC,
             num_subcores=NS)


def _sc_deg(dst_rows, zinit):
    @functools.partial(
        pl.kernel,
        out_type=jax.ShapeDtypeStruct((NC, NA), jnp.float32),
        mesh=plsc.VectorSubcoreMesh(**_MESH),
        scratch_types=[
            pltpu.VMEM((KI, B), jnp.int32),
            pltpu.VMEM((B,), jnp.float32),
            pltpu.VMEM_SHARED((NA,), jnp.float32),
            pltpu.SemaphoreType.DMA,
        ],
    )
    def k(dst_h, z_h, out_h, idx_v, ones_v, acc, sem_s):
        c = lax.axis_index("c")
        s = lax.axis_index("s")
        w = c * NS + s
        for t in range(B // 16):
            ones_v[pl.ds(t * 16, 16)] = jnp.ones((16,), jnp.float32)
        pltpu.sync_copy(z_h.at[pl.ds(s * SL, SL)], acc.at[pl.ds(s * SL, SL)])
        plsc.subcore_barrier()

        def body(i, carry):
            rbase = w * RW + i * KI
            pltpu.sync_copy(dst_h.at[pl.ds(rbase, KI)], idx_v)
            descs = [
                pltpu.async_copy(ones_v, acc.at[idx_v.at[j]], sem_s, add=True)
                for j in range(KI)
            ]
            for d in descs:
                d.wait()
            return carry

        lax.fori_loop(0, NBODY, body, 0)
        plsc.subcore_barrier()
        pltpu.sync_copy(acc.at[pl.ds(s * SL, SL)],
                        out_h.at[c, pl.ds(s * SL, SL)])

    return k(dst_rows, zinit)


def _sc_agg1(src_rows, dst_rows, table, zinit):
    @functools.partial(
        pl.kernel,
        out_type=jax.ShapeDtypeStruct((NC, NA), jnp.float32),
        mesh=plsc.VectorSubcoreMesh(**_MESH),
        scratch_types=[
            pltpu.VMEM((NA,), jnp.float32),
            pltpu.VMEM((KI, B), jnp.int32),
            pltpu.VMEM((KI, B), jnp.int32),
            pltpu.VMEM((KI, B), jnp.int32),
            pltpu.VMEM((KI, B), jnp.int32),
            pltpu.VMEM((KI, B), jnp.float32),
            pltpu.VMEM_SHARED((NA,), jnp.float32),
            pltpu.SemaphoreType.DMA,
            pltpu.SemaphoreType.DMA,
            pltpu.SemaphoreType.DMA,
        ],
        compiler_params=pltpu.CompilerParams(needs_layout_passes=False),
    )
    def k(src_h, dst_h, tab_h, z_h, out_h, tab_v, src_a, dst_a, src_b, dst_b,
          vals_v, acc, sem_la, sem_lb, sem_s):
        c = lax.axis_index("c")
        s = lax.axis_index("s")
        w = c * NS + s
        pltpu.sync_copy(tab_h, tab_v)
        pltpu.sync_copy(z_h.at[pl.ds(s * SL, SL)], acc.at[pl.ds(s * SL, SL)])
        plsc.subcore_barrier()

        def fire(i, sv, dv, sem):
            rbase = w * RW + i * KI
            pltpu.async_copy(src_h.at[pl.ds(rbase, KI)], sv, sem)
            pltpu.async_copy(dst_h.at[pl.ds(rbase, KI)], dv, sem)

        def wait(sv, dv, sem):
            pltpu.make_async_copy(src_h.at[pl.ds(0, KI)], sv, sem).wait()
            pltpu.make_async_copy(dst_h.at[pl.ds(0, KI)], dv, sem).wait()

        def process(sv, dv):
            sd = []
            for j in range(KI):
                for t in range(B // 16):
                    sl = (j, pl.ds(t * 16, 16))
                    vals_v[sl] = plsc.load_gather(tab_v, [sv[sl]])
                sd.append(pltpu.async_copy(vals_v.at[j], acc.at[dv.at[j]],
                                           sem_s, add=True))
            for d in sd:
                d.wait()

        fire(0, src_a, dst_a, sem_la)

        def body(g, carry):
            fire(2 * g + 1, src_b, dst_b, sem_lb)
            wait(src_a, dst_a, sem_la)
            process(src_a, dst_a)
            fire(2 * g + 2, src_a, dst_a, sem_la)
            wait(src_b, dst_b, sem_lb)
            process(src_b, dst_b)
            return carry

        lax.fori_loop(0, NBODY // 2, body, 0)
        wait(src_a, dst_a, sem_la)
        process(src_a, dst_a)
        plsc.subcore_barrier()
        pltpu.sync_copy(acc.at[pl.ds(s * SL, SL)],
                        out_h.at[c, pl.ds(s * SL, SL)])

    return k(src_rows, dst_rows, table, zinit)


EW2 = EP // 16
RW2 = EW2 // B
KI2 = 16
NBODY2 = RW2 // KI2
SL2 = 2 * NA // NS


def _sc_agg2(src_rows, dst_rows, table2, zinit):
    @functools.partial(
        pl.kernel,
        out_type=jax.ShapeDtypeStruct((NC, 2 * NA), jnp.float32),
        mesh=plsc.VectorSubcoreMesh(**_MESH),
        scratch_types=[
            pltpu.VMEM((NA,), jnp.float32),
            pltpu.VMEM((KI2, B), jnp.int32),
            pltpu.VMEM((KI2, B), jnp.int32),
            pltpu.VMEM((KI2, B), jnp.int32),
            pltpu.VMEM((KI2, B), jnp.int32),
            pltpu.VMEM((KI2, B), jnp.float32),
            pltpu.VMEM_SHARED((2 * NA,), jnp.float32),
            pltpu.SemaphoreType.DMA,
            pltpu.SemaphoreType.DMA,
            pltpu.SemaphoreType.DMA,
        ],
        compiler_params=pltpu.CompilerParams(needs_layout_passes=False),
    )
    def k(src_h, dst_h, tab_h, z_h, out_h, tab_v, src_a, dst_a, src_b, dst_b,
          vals_v, acc, sem_la, sem_lb, sem_s):
        c = lax.axis_index("c")
        s = lax.axis_index("s")
        ch = s % 2
        chunk = c * 8 + s // 2
        offs = ch * NA
        offv = jnp.full((16,), offs, jnp.int32)
        pltpu.sync_copy(tab_h.at[pl.ds(ch * NA, NA)], tab_v)
        pltpu.sync_copy(z_h.at[pl.ds(0, SL2)], acc.at[pl.ds(s * SL2, SL2)])
        plsc.subcore_barrier()

        def fire(i, sv, dv, sem):
            rbase = jnp.minimum(chunk * RW2 + i * KI2, EPR - KI2)
            pltpu.async_copy(src_h.at[pl.ds(rbase, KI2)], sv, sem)
            pltpu.async_copy(dst_h.at[pl.ds(rbase, KI2)], dv, sem)

        def wait(sv, dv, sem):
            pltpu.make_async_copy(src_h.at[pl.ds(0, KI2)], sv, sem).wait()
            pltpu.make_async_copy(dst_h.at[pl.ds(0, KI)], dv, sem).wait()

        def process(sv, dv):
            sd = []
            for j in range(KI2):
                for t in range(B // 16):
                    sl = (j, pl.ds(t * 16, 16))
                    dv[sl] = dv[sl] + offv
                    vals_v[sl] = plsc.load_gather(tab_v, [sv[sl]])
                sd.append(pltpu.async_copy(vals_v.at[j], acc.at[dv.at[j]],
                                           sem_s, add=True))
            for d in sd:
                d.wait()

        fire(0, src_a, dst_a, sem_la)

        def body(g, carry):
            fire(2 * g + 1, src_b, dst_b, sem_lb)
            wait(src_a, dst_a, sem_la)
            process(src_a, dst_a)
            fire(2 * g + 2, src_a, dst_a, sem_la)
            wait(src_b, dst_b, sem_lb)
            process(src_b, dst_b)
            return carry

        lax.fori_loop(0, NBODY2 // 2, body, 0)
        wait(src_a, dst_a, sem_la)
        plsc.subcore_barrier()
        pltpu.sync_copy(acc.at[pl.ds(s * SL2, SL2)],
                        out_h.at[c, pl.ds(s * SL2, SL2)])

    return k(src_rows, dst_rows, table2, zinit)


def _tc_prep(degcm, xc):
    def body(deg_ref, xc_ref, dis_ref, u_ref):
        deg = deg_ref[0] + deg_ref[1] + 1.0
        y = lax.rsqrt(deg)
        dis = y * (1.5 - 0.5 * deg * y * y)
        dis_ref[...] = dis
        u_ref[0] = dis * xc_ref[0]
        u_ref[1] = dis * xc_ref[1]

    return pl.pallas_call(
        body,
        out_shape=(jax.ShapeDtypeStruct((ROWS, 128), jnp.float32),
                   jax.ShapeDtypeStruct((2, ROWS, 128), jnp.float32)),
    )(degcm, xc)


def _tc_mid(s1a, s1b, ucm, discm, W1, b1, W2):
    def body(s1a_ref, s1b_ref, u_ref, dis_ref, w1_ref, b1_ref, w2_ref, v_ref):
        dis = dis_ref[...]
        za = dis * (s1a_ref[0] + s1a_ref[1] + u_ref[0])
        zb = dis * (s1b_ref[0] + s1b_ref[1] + u_ref[1])
        t = jnp.zeros((ROWS, 128), jnp.float32)
        for o in range(16):
            h = jnp.maximum(za * w1_ref[0, o] + zb * w1_ref[1, o] + b1_ref[o],
                            0.0)
            t = t + h * w2_ref[o, 0]
        v_ref[...] = dis * t

    vspec = pl.BlockSpec(memory_space=pltpu.VMEM)
    sspec = pl.BlockSpec(memory_space=pltpu.SMEM)
    return pl.pallas_call(
        body,
        in_specs=[vspec, vspec, vspec, vspec, sspec, sspec, sspec],
        out_specs=vspec,
        out_shape=jax.ShapeDtypeStruct((ROWS, 128), jnp.float32),
    )(s1a, s1b, ucm, discm, W1, b1, W2)


def _tc_fin(s2cm, vcm, discm, b2):
    def body(s2_ref, v_ref, dis_ref, b2_ref, o_ref):
        o_ref[...] = (dis_ref[...] * (s2_ref[0] + s2_ref[1] + v_ref[...])
                      + b2_ref[0])

    vspec = pl.BlockSpec(memory_space=pltpu.VMEM)
    sspec = pl.BlockSpec(memory_space=pltpu.SMEM)
    return pl.pallas_call(
        body,
        in_specs=[vspec, vspec, vspec, sspec],
        out_specs=vspec,
        out_shape=jax.ShapeDtypeStruct((ROWS, 128), jnp.float32),
    )(s2cm, vcm, discm, b2)


def kernel(x, edge_index, W1, b1, W2, b2):
    ei = edge_index.astype(jnp.int32)
    npad = EP - E
    padv = (jnp.arange(npad, dtype=jnp.int32) % 2048) + N
    src_rows = jnp.concatenate([ei[0], padv]).reshape(EPR, B)
    dst_rows = jnp.concatenate([ei[1], padv]).reshape(EPR, B)


    xp = jnp.pad(x, ((0, NA - N), (0, 0)))
    xc = xp.T.reshape(2, ROWS, 128)
    z1 = jnp.zeros((NA,), jnp.float32)

    degp = _sc_deg(dst_rows, z1)
    discm, ucm = _tc_prep(degp.reshape(NC, ROWS, 128), xc)

    u_flat = ucm.reshape(2 * NA)
    s1p = _sc_agg2(src_rows, dst_rows, u_flat, z1)
    s1f = s1p.reshape(NC, 2, ROWS, 128)
    vcm = _tc_mid(s1f[:, 0], s1f[:, 1],
                  ucm, discm, W1, b1, W2)

    v_tab = vcm.reshape(NA)
    s2p = _sc_agg1(src_rows, dst_rows, v_tab, z1)
    outcm = _tc_fin(s2p.reshape(NC, ROWS, 128), vcm, discm, b2)
    return outcm.reshape(NA)[:N]


# speedup vs baseline: 394.8154x; 1.0750x over previous
import functools

import jax
import jax.numpy as jnp
from jax import lax
from jax.experimental import pallas as pl
from jax.experimental.pallas import tpu as pltpu
from jax.experimental.pallas import tpu_sc as plsc

N = 100000
E = 6400000

NC = 2
NS = 16
NW = NC * NS

B = 128
KI = 32
CL = B * KI

NA = 102400
SL = NA // NS
ROWS = NA // 128

EW = 200704
EP = EW * NW
RW = EW // B
NBODY = EW // CL
EPR = EP // B

_MESH = dict(core_axis_name="c", subcore_axis_name="s", num_cores=NC,
             num_subcores=NS)


def _sc_deg(dst_rows, zinit):
    @functools.partial(
        pl.kernel,
        out_type=jax.ShapeDtypeStruct((NC, NA), jnp.float32),
        mesh=plsc.VectorSubcoreMesh(**_MESH),
        scratch_types=[
            pltpu.VMEM((KI, B), jnp.int32),
            pltpu.VMEM((KI, B), jnp.int32),
            pltpu.VMEM((B,), jnp.float32),
            pltpu.VMEM_SHARED((NA,), jnp.float32),
            pltpu.SemaphoreType.DMA,
            pltpu.SemaphoreType.DMA,
            pltpu.SemaphoreType.DMA,
        ],
    )
    def k(dst_h, z_h, out_h, idx_a, idx_b, ones_v, acc, sem_la, sem_lb,
          sem_s):
        c = lax.axis_index("c")
        s = lax.axis_index("s")
        w = c * NS + s
        for t in range(B // 16):
            ones_v[pl.ds(t * 16, 16)] = jnp.ones((16,), jnp.float32)
        pltpu.sync_copy(z_h.at[pl.ds(s * SL, SL)], acc.at[pl.ds(s * SL, SL)])
        plsc.subcore_barrier()

        def fire(i, dv, sem):
            rbase = w * RW + i * KI
            pltpu.async_copy(dst_h.at[pl.ds(rbase, KI)], dv, sem)

        def wait(dv, sem):
            pltpu.make_async_copy(dst_h.at[pl.ds(0, KI)], dv, sem).wait()

        def process(dv):
            sd = [
                pltpu.async_copy(ones_v, acc.at[dv.at[j]], sem_s, add=True)
                for j in range(KI)
            ]
            for d in sd:
                d.wait()

        fire(0, idx_a, sem_la)

        def body(g, carry):
            fire(2 * g + 1, idx_b, sem_lb)
            wait(idx_a, sem_la)
            process(idx_a)
            fire(2 * g + 2, idx_a, sem_la)
            wait(idx_b, sem_lb)
            process(idx_b)
            return carry

        lax.fori_loop(0, NBODY // 2, body, 0)
        wait(idx_a, sem_la)
        process(idx_a)
        plsc.subcore_barrier()
        pltpu.sync_copy(acc.at[pl.ds(s * SL, SL)],
                        out_h.at[c, pl.ds(s * SL, SL)])

    return k(dst_rows, zinit)


def _sc_agg1(src_rows, dst_rows, table, zinit):
    @functools.partial(
        pl.kernel,
        out_type=jax.ShapeDtypeStruct((NC, NA), jnp.float32),
        mesh=plsc.VectorSubcoreMesh(**_MESH),
        scratch_types=[
            pltpu.VMEM((NA,), jnp.float32),
            pltpu.VMEM((KI, B), jnp.int32),
            pltpu.VMEM((KI, B), jnp.int32),
            pltpu.VMEM((KI, B), jnp.int32),
            pltpu.VMEM((KI, B), jnp.int32),
            pltpu.VMEM((KI, B), jnp.float32),
            pltpu.VMEM_SHARED((NA,), jnp.float32),
            pltpu.SemaphoreType.DMA,
            pltpu.SemaphoreType.DMA,
            pltpu.SemaphoreType.DMA,
        ],
        compiler_params=pltpu.CompilerParams(needs_layout_passes=False),
    )
    def k(src_h, dst_h, tab_h, z_h, out_h, tab_v, src_a, dst_a, src_b, dst_b,
          vals_v, acc, sem_la, sem_lb, sem_s):
        c = lax.axis_index("c")
        s = lax.axis_index("s")
        w = c * NS + s
        pltpu.sync_copy(tab_h, tab_v)
        pltpu.sync_copy(z_h.at[pl.ds(s * SL, SL)], acc.at[pl.ds(s * SL, SL)])
        plsc.subcore_barrier()

        def fire(i, sv, dv, sem):
            rbase = w * RW + i * KI
            pltpu.async_copy(src_h.at[pl.ds(rbase, KI)], sv, sem)
            pltpu.async_copy(dst_h.at[pl.ds(rbase, KI)], dv, sem)

        def wait(sv, dv, sem):
            pltpu.make_async_copy(src_h.at[pl.ds(0, KI)], sv, sem).wait()
            pltpu.make_async_copy(dst_h.at[pl.ds(0, KI)], dv, sem).wait()

        def process(sv, dv):
            sd = []
            for j in range(KI):
                for t in range(B // 16):
                    sl = (j, pl.ds(t * 16, 16))
                    vals_v[sl] = plsc.load_gather(tab_v, [sv[sl]])
                sd.append(pltpu.async_copy(vals_v.at[j], acc.at[dv.at[j]],
                                           sem_s, add=True))
            for d in sd:
                d.wait()

        fire(0, src_a, dst_a, sem_la)

        def body(g, carry):
            fire(2 * g + 1, src_b, dst_b, sem_lb)
            wait(src_a, dst_a, sem_la)
            process(src_a, dst_a)
            fire(2 * g + 2, src_a, dst_a, sem_la)
            wait(src_b, dst_b, sem_lb)
            process(src_b, dst_b)
            return carry

        lax.fori_loop(0, NBODY // 2, body, 0)
        wait(src_a, dst_a, sem_la)
        process(src_a, dst_a)
        plsc.subcore_barrier()
        pltpu.sync_copy(acc.at[pl.ds(s * SL, SL)],
                        out_h.at[c, pl.ds(s * SL, SL)])

    return k(src_rows, dst_rows, table, zinit)


EW2 = EP // 16
RW2 = EW2 // B
KI2 = 16
NBODY2 = RW2 // KI2
SL2 = 2 * NA // NS


def _sc_agg2(src_rows, dst_rows, table2, zinit):
    @functools.partial(
        pl.kernel,
        out_type=jax.ShapeDtypeStruct((NC, 2 * NA), jnp.float32),
        mesh=plsc.VectorSubcoreMesh(**_MESH),
        scratch_types=[
            pltpu.VMEM((NA,), jnp.float32),
            pltpu.VMEM((KI2, B), jnp.int32),
            pltpu.VMEM((KI2, B), jnp.int32),
            pltpu.VMEM((KI2, B), jnp.int32),
            pltpu.VMEM((KI2, B), jnp.int32),
            pltpu.VMEM((KI2, B), jnp.float32),
            pltpu.VMEM_SHARED((2 * NA,), jnp.float32),
            pltpu.SemaphoreType.DMA,
            pltpu.SemaphoreType.DMA,
            pltpu.SemaphoreType.DMA,
        ],
        compiler_params=pltpu.CompilerParams(needs_layout_passes=False),
    )
    def k(src_h, dst_h, tab_h, z_h, out_h, tab_v, src_a, dst_a, src_b, dst_b,
          vals_v, acc, sem_la, sem_lb, sem_s):
        c = lax.axis_index("c")
        s = lax.axis_index("s")
        ch = s % 2
        chunk = c * 8 + s // 2
        offs = ch * NA
        offv = jnp.full((16,), offs, jnp.int32)
        pltpu.sync_copy(tab_h.at[pl.ds(ch * NA, NA)], tab_v)
        pltpu.sync_copy(z_h.at[pl.ds(0, SL2)], acc.at[pl.ds(s * SL2, SL2)])
        plsc.subcore_barrier()

        def fire(i, sv, dv, sem):
            rbase = jnp.minimum(chunk * RW2 + i * KI2, EPR - KI2)
            pltpu.async_copy(src_h.at[pl.ds(rbase, KI2)], sv, sem)
            pltpu.async_copy(dst_h.at[pl.ds(rbase, KI2)], dv, sem)

        def wait(sv, dv, sem):
            pltpu.make_async_copy(src_h.at[pl.ds(0, KI2)], sv, sem).wait()
            pltpu.make_async_copy(dst_h.at[pl.ds(0, KI)], dv, sem).wait()

        def process(sv, dv):
            sd = []
            for j in range(KI2):
                for t in range(B // 16):
                    sl = (j, pl.ds(t * 16, 16))
                    dv[sl] = dv[sl] + offv
                    vals_v[sl] = plsc.load_gather(tab_v, [sv[sl]])
                sd.append(pltpu.async_copy(vals_v.at[j], acc.at[dv.at[j]],
                                           sem_s, add=True))
            for d in sd:
                d.wait()

        fire(0, src_a, dst_a, sem_la)

        def body(g, carry):
            fire(2 * g + 1, src_b, dst_b, sem_lb)
            wait(src_a, dst_a, sem_la)
            process(src_a, dst_a)
            fire(2 * g + 2, src_a, dst_a, sem_la)
            wait(src_b, dst_b, sem_lb)
            process(src_b, dst_b)
            return carry

        lax.fori_loop(0, NBODY2 // 2, body, 0)
        wait(src_a, dst_a, sem_la)
        plsc.subcore_barrier()
        pltpu.sync_copy(acc.at[pl.ds(s * SL2, SL2)],
                        out_h.at[c, pl.ds(s * SL2, SL2)])

    return k(src_rows, dst_rows, table2, zinit)


def _tc_prep(degcm, xc):
    def body(deg_ref, xc_ref, dis_ref, u_ref):
        deg = deg_ref[0] + deg_ref[1] + 1.0
        y = lax.rsqrt(deg)
        dis = y * (1.5 - 0.5 * deg * y * y)
        dis_ref[...] = dis
        u_ref[0] = dis * xc_ref[0]
        u_ref[1] = dis * xc_ref[1]

    return pl.pallas_call(
        body,
        out_shape=(jax.ShapeDtypeStruct((ROWS, 128), jnp.float32),
                   jax.ShapeDtypeStruct((2, ROWS, 128), jnp.float32)),
    )(degcm, xc)


def _tc_mid(s1a, s1b, ucm, discm, W1, b1, W2):
    def body(s1a_ref, s1b_ref, u_ref, dis_ref, w1_ref, b1_ref, w2_ref, v_ref):
        dis = dis_ref[...]
        za = dis * (s1a_ref[0] + s1a_ref[1] + u_ref[0])
        zb = dis * (s1b_ref[0] + s1b_ref[1] + u_ref[1])
        t = jnp.zeros((ROWS, 128), jnp.float32)
        for o in range(16):
            h = jnp.maximum(za * w1_ref[0, o] + zb * w1_ref[1, o] + b1_ref[o],
                            0.0)
            t = t + h * w2_ref[o, 0]
        v_ref[...] = dis * t

    vspec = pl.BlockSpec(memory_space=pltpu.VMEM)
    sspec = pl.BlockSpec(memory_space=pltpu.SMEM)
    return pl.pallas_call(
        body,
        in_specs=[vspec, vspec, vspec, vspec, sspec, sspec, sspec],
        out_specs=vspec,
        out_shape=jax.ShapeDtypeStruct((ROWS, 128), jnp.float32),
    )(s1a, s1b, ucm, discm, W1, b1, W2)


def _tc_fin(s2cm, vcm, discm, b2):
    def body(s2_ref, v_ref, dis_ref, b2_ref, o_ref):
        o_ref[...] = (dis_ref[...] * (s2_ref[0] + s2_ref[1] + v_ref[...])
                      + b2_ref[0])

    vspec = pl.BlockSpec(memory_space=pltpu.VMEM)
    sspec = pl.BlockSpec(memory_space=pltpu.SMEM)
    return pl.pallas_call(
        body,
        in_specs=[vspec, vspec, vspec, sspec],
        out_specs=vspec,
        out_shape=jax.ShapeDtypeStruct((ROWS, 128), jnp.float32),
    )(s2cm, vcm, discm, b2)


def kernel(x, edge_index, W1, b1, W2, b2):
    ei = edge_index.astype(jnp.int32)
    npad = EP - E
    padv = (jnp.arange(npad, dtype=jnp.int32) % 2048) + N
    src_rows = jnp.concatenate([ei[0], padv]).reshape(EPR, B)
    dst_rows = jnp.concatenate([ei[1], padv]).reshape(EPR, B)


    xp = jnp.pad(x, ((0, NA - N), (0, 0)))
    xc = xp.T.reshape(2, ROWS, 128)
    z1 = jnp.zeros((NA,), jnp.float32)

    degp = _sc_deg(dst_rows, z1)
    discm, ucm = _tc_prep(degp.reshape(NC, ROWS, 128), xc)

    u_flat = ucm.reshape(2 * NA)
    s1p = _sc_agg2(src_rows, dst_rows, u_flat, z1)
    s1f = s1p.reshape(NC, 2, ROWS, 128)
    vcm = _tc_mid(s1f[:, 0], s1f[:, 1],
                  ucm, discm, W1, b1, W2)

    v_tab = vcm.reshape(NA)
    s2p = _sc_agg1(src_rows, dst_rows, v_tab, z1)
    outcm = _tc_fin(s2p.reshape(NC, ROWS, 128), vcm, discm, b2)
    return outcm.reshape(NA)[:N]
